# Initial kernel scaffold; baseline (speedup 1.0000x reference)
import jax
import jax.numpy as jnp
from jax.experimental import pallas as pl


def kernel(x, edge_index, W0, b0, W1, b1, W2, b2):
    raise NotImplementedError("write your pallas kernel here")


# speedup vs baseline: 1.3126x; 1.3126x over previous
import functools
import jax
import jax.numpy as jnp
from jax.experimental import pallas as pl

N_PAD = 10240
ROW_BLK = 1024


def _mm_a_body(x_ref, w_ref, dinv_ref, o_ref):
    o_ref[...] = jnp.dot(x_ref[...], w_ref[...],
                         preferred_element_type=jnp.float32) * dinv_ref[...]


def _mm_b_body(g_ref, dinv_ref, b_ref, w_ref, y_ref, h_ref):
    y = jnp.maximum(g_ref[...] * dinv_ref[...] + b_ref[...], 0.0)
    y_ref[...] = y
    h_ref[...] = jnp.dot(y, w_ref[...],
                         preferred_element_type=jnp.float32) * dinv_ref[...]


def _mm_br_body(g_ref, dinv_ref, b_ref, r_ref, w_ref, y_ref, h_ref):
    y = jnp.maximum(r_ref[...] + g_ref[...] * dinv_ref[...] + b_ref[...], 0.0)
    y_ref[...] = y
    h_ref[...] = jnp.dot(y, w_ref[...],
                         preferred_element_type=jnp.float32) * dinv_ref[...]


def _ew_body(g_ref, dinv_ref, b_ref, r_ref, y_ref):
    y_ref[...] = jnp.maximum(
        r_ref[...] + g_ref[...] * dinv_ref[...] + b_ref[...], 0.0)


def _row_spec(width):
    return pl.BlockSpec((ROW_BLK, width), lambda i: (i, 0))


def _full_spec(shape):
    return pl.BlockSpec(shape, lambda i: (0, 0))


def _mm_a(x, w, dinv):
    k = x.shape[1]
    n = w.shape[1]
    return pl.pallas_call(
        _mm_a_body,
        grid=(N_PAD // ROW_BLK,),
        in_specs=[_row_spec(k), _full_spec((k, n)), _row_spec(1)],
        out_specs=_row_spec(n),
        out_shape=jax.ShapeDtypeStruct((N_PAD, n), jnp.float32),
    )(x, w, dinv)


def _mm_b(g, dinv, b, w):
    n = w.shape[1]
    k = g.shape[1]
    return pl.pallas_call(
        _mm_b_body,
        grid=(N_PAD // ROW_BLK,),
        in_specs=[_row_spec(k), _row_spec(1), _full_spec((1, k)),
                  _full_spec((k, n))],
        out_specs=[_row_spec(k), _row_spec(n)],
        out_shape=[jax.ShapeDtypeStruct((N_PAD, k), jnp.float32),
                   jax.ShapeDtypeStruct((N_PAD, n), jnp.float32)],
    )(g, dinv, b.reshape(1, k), w)


def _mm_br(g, dinv, b, resid, w):
    n = w.shape[1]
    k = g.shape[1]
    return pl.pallas_call(
        _mm_br_body,
        grid=(N_PAD // ROW_BLK,),
        in_specs=[_row_spec(k), _row_spec(1), _full_spec((1, k)),
                  _row_spec(k), _full_spec((k, n))],
        out_specs=[_row_spec(k), _row_spec(n)],
        out_shape=[jax.ShapeDtypeStruct((N_PAD, k), jnp.float32),
                   jax.ShapeDtypeStruct((N_PAD, n), jnp.float32)],
    )(g, dinv, b.reshape(1, k), resid, w)


def _ew(g, dinv, b, resid):
    k = g.shape[1]
    return pl.pallas_call(
        _ew_body,
        grid=(N_PAD // ROW_BLK,),
        in_specs=[_row_spec(k), _row_spec(1), _full_spec((1, k)),
                  _row_spec(k)],
        out_specs=_row_spec(k),
        out_shape=jax.ShapeDtypeStruct((N_PAD, k), jnp.float32),
    )(g, dinv, b.reshape(1, k), resid)


def _propagate(hp, src, dst):
    agg = hp.at[dst].add(hp[src], mode="drop")
    return agg


def kernel(x, edge_index, W0, b0, W1, b1, W2, b2):
    n = x.shape[0]
    src = edge_index[0].astype(jnp.int32)
    dst = edge_index[1].astype(jnp.int32)

    deg = jnp.zeros((n,), jnp.float32).at[dst].add(1.0) + 1.0
    dinv = jax.lax.rsqrt(deg)
    dinv = jnp.pad(dinv, (0, N_PAD - n)).reshape(N_PAD, 1)
    xp = jnp.pad(x, ((0, N_PAD - n), (0, 0)))

    hp = _mm_a(xp, W0, dinv)
    g = _propagate(hp, src, dst)
    y0, hp = _mm_b(g, dinv, b0, W1[0])
    g = _propagate(hp, src, dst)
    _, hp = _mm_b(g, dinv, b1[0], W2[0])
    g = _propagate(hp, src, dst)
    y1, hp = _mm_br(g, dinv, b2[0], y0, W1[1])
    g = _propagate(hp, src, dst)
    _, hp = _mm_b(g, dinv, b1[1], W2[1])
    g = _propagate(hp, src, dst)
    y2, hp = _mm_br(g, dinv, b2[1], y1, W1[2])
    g = _propagate(hp, src, dst)
    _, hp = _mm_b(g, dinv, b1[2], W2[2])
    g = _propagate(hp, src, dst)
    y3 = _ew(g, dinv, b2[2], y2)

    return (y1[:n], y2[:n], y3[:n])
